# baseline (speedup 1.0000x reference)
import functools

import jax
import jax.numpy as jnp
from jax import lax
from jax.experimental import pallas as pl
from jax.experimental.pallas import tpu as pltpu
from jax.experimental.pallas import tpu_sc as plsc

D = 32
H = 32
L = 16
T = 8
G3 = 128
PK = 4
GW = G3 * PK

NC = 2
NS = 16
NW = NC * NS
GSZ = 125
CHUNK_G = 8
CHUNK = GSZ * CHUNK_G

_MESH = plsc.VectorSubcoreMesh(core_axis_name="c", subcore_axis_name="s")


def _sc_gather(table, idx_groups, E):
    n_groups = idx_groups.shape[0]
    gpw = n_groups // NW
    chunks = gpw // CHUNK_G

    @functools.partial(
        pl.kernel,
        out_type=jax.ShapeDtypeStruct((E, D), jnp.bfloat16),
        mesh=_MESH,
        scratch_types=[
            pltpu.VMEM((gpw, GSZ), jnp.int32),
            pltpu.VMEM((CHUNK, D), jnp.bfloat16),
            pltpu.VMEM((CHUNK, D), jnp.bfloat16),
            pltpu.SemaphoreType.DMA,
            pltpu.SemaphoreType.DMA,
            pltpu.SemaphoreType.DMA,
            pltpu.SemaphoreType.DMA,
        ],
        compiler_params=pltpu.CompilerParams(use_tc_tiling_on_sc=False),
    )
    def k(table_hbm, idx_hbm, out_hbm, idx_v, rows0, rows1,
          gsem0, gsem1, wsem0, wsem1):
        cid = lax.axis_index("c")
        sid = lax.axis_index("s")
        wid = sid * NC + cid
        obase = wid * gpw * GSZ
        pltpu.sync_copy(idx_hbm.at[pl.ds(wid * gpw, gpw)], idx_v)
        bufs = (rows0, rows1)
        gsems = (gsem0, gsem1)
        wsems = (wsem0, wsem1)

        def fire(c, buf, sem):
            for g in range(CHUNK_G):
                pltpu.async_copy(table_hbm.at[idx_v.at[c * CHUNK_G + g]],
                                 buf.at[pl.ds(g * GSZ, GSZ)], sem)

        def drain(buf, sem):
            pltpu.make_async_copy(out_hbm.at[pl.ds(0, CHUNK)], buf,
                                  sem).wait()

        def drain_write(buf, sem):
            pltpu.make_async_copy(buf, out_hbm.at[pl.ds(0, CHUNK)],
                                  sem).wait()

        fire(0, bufs[0], gsems[0])

        def body(c2, _):
            for b in range(2):
                c = 2 * c2 + b
                o = 1 - b
                nxt = c + 1

                @pl.when(nxt < chunks)
                def _():
                    @pl.when(nxt >= 2)
                    def _():
                        drain_write(bufs[o], wsems[o])
                    fire(nxt, bufs[o], gsems[o])

                drain(bufs[b], gsems[b])
                pltpu.async_copy(bufs[b], out_hbm.at[pl.ds(obase + c * CHUNK,
                                                           CHUNK)], wsems[b])
            return 0

        lax.fori_loop(0, chunks // 2, body, 0)
        drain_write(bufs[0], wsems[0])
        drain_write(bufs[1], wsems[1])

    return k(table, idx_groups)


def _sc_scatter_add(msgs, idx_groups, zeros, n_links):
    n_groups = idx_groups.shape[0]
    gpw = n_groups // NW
    chunks = gpw // CHUNK_G
    rows_per_sub = n_links // NS

    @functools.partial(
        pl.kernel,
        out_type=jax.ShapeDtypeStruct((NC, n_links, D), jnp.float32),
        mesh=_MESH,
        scratch_types=[
            pltpu.VMEM((gpw, GSZ), jnp.int32),
            pltpu.VMEM((CHUNK, D), jnp.float32),
            pltpu.VMEM((CHUNK, D), jnp.float32),
            pltpu.VMEM_SHARED((n_links, D), jnp.float32),
            pltpu.SemaphoreType.DMA,
            pltpu.SemaphoreType.DMA,
            pltpu.SemaphoreType.DMA,
            pltpu.SemaphoreType.DMA,
        ],
        compiler_params=pltpu.CompilerParams(use_tc_tiling_on_sc=False),
    )
    def k(msgs_hbm, idx_hbm, zeros_hbm, out_hbm, idx_v, rows0, rows1, acc_sh,
          rsem0, rsem1, ssem0, ssem1):
        cid = lax.axis_index("c")
        sid = lax.axis_index("s")
        wid = sid * NC + cid
        pltpu.sync_copy(zeros_hbm.at[pl.ds(sid * rows_per_sub, rows_per_sub)],
                        acc_sh.at[pl.ds(sid * rows_per_sub, rows_per_sub)])
        pltpu.sync_copy(idx_hbm.at[pl.ds(wid * gpw, gpw)], idx_v)
        plsc.subcore_barrier()
        bufs = (rows0, rows1)
        rsems = (rsem0, rsem1)
        ssems = (ssem0, ssem1)
        mbase = wid * gpw * GSZ

        def fire_sadds(c, buf, sem):
            for g in range(CHUNK_G):
                pltpu.async_copy(buf.at[pl.ds(g * GSZ, GSZ)],
                                 acc_sh.at[idx_v.at[c * CHUNK_G + g]],
                                 sem, add=True)

        def drain_chunk(buf, sem):
            pltpu.make_async_copy(msgs_hbm.at[pl.ds(0, CHUNK)], buf,
                                  sem).wait()

        pltpu.async_copy(msgs_hbm.at[pl.ds(mbase, CHUNK)], bufs[0], rsems[0])

        def body(c2, _):
            for b in range(2):
                c = 2 * c2 + b
                o = 1 - b
                nxt = c + 1

                @pl.when(nxt < chunks)
                def _():
                    @pl.when(nxt >= 2)
                    def _():
                        drain_chunk(bufs[o], ssems[o])
                    pltpu.async_copy(
                        msgs_hbm.at[pl.ds(mbase + nxt * CHUNK, CHUNK)],
                        bufs[o], rsems[o])

                drain_chunk(bufs[b], rsems[b])
                fire_sadds(c, bufs[b], ssems[b])
            return 0

        lax.fori_loop(0, chunks // 2, body, 0)
        drain_chunk(bufs[0], ssems[0])
        drain_chunk(bufs[1], ssems[1])
        plsc.subcore_barrier()
        pltpu.sync_copy(acc_sh.at[pl.ds(sid * rows_per_sub, rows_per_sub)],
                        out_hbm.at[cid, pl.ds(sid * rows_per_sub,
                                              rows_per_sub)])

    return k(msgs, idx_groups, zeros)


def _pack_gru_weights(Wih, Whh, bih, bhh):
    Wx = jnp.concatenate(
        [Wih[0:H].T, Wih[H:2 * H].T, Wih[2 * H:3 * H].T,
         jnp.zeros((D, H), Wih.dtype)], axis=1)
    Wh = jnp.concatenate(
        [Whh[0:H].T, Whh[H:2 * H].T, jnp.zeros((H, H), Whh.dtype),
         Whh[2 * H:3 * H].T], axis=1)
    b = jnp.concatenate(
        [bih[0:H] + bhh[0:H], bih[H:2 * H] + bhh[H:2 * H],
         bih[2 * H:3 * H], bhh[2 * H:3 * H]])
    eye = jnp.eye(PK, dtype=Wx.dtype)
    Wx4 = jnp.einsum('rgc,jk->jrgkc', Wx.reshape(D, 4, H), eye)
    Wx4 = Wx4.reshape(PK * D, GW)
    Wh4 = jnp.einsum('rgc,jk->jrgkc', Wh.reshape(H, 4, H), eye)
    Wh4 = Wh4.reshape(PK * H, GW)
    b4 = jnp.tile(b.reshape(4, 1, H), (1, PK, 1)).reshape(1, GW)
    return Wx4, Wh4, b4


def _gru_gates4(g4, h4):
    r = jax.nn.sigmoid(g4[:, 0:G3])
    z = jax.nn.sigmoid(g4[:, G3:2 * G3])
    n = jnp.tanh(g4[:, 2 * G3:3 * G3] + r * g4[:, 3 * G3:4 * G3])
    return (1.0 - z) * n + z * h4


def _path_scan_body(xg_ref, h0_ref, wc_ref, b_ref, hs_ref):
    h = h0_ref[...]
    wc = wc_ref[...].astype(jnp.bfloat16)
    b = b_ref[...]
    for s in range(L):
        xh = jnp.concatenate(
            [xg_ref[s].astype(jnp.bfloat16), h.astype(jnp.bfloat16)], axis=1)
        g = jnp.dot(xh, wc, preferred_element_type=jnp.float32) + b
        h = _gru_gates4(g, h)
        hs_ref[s] = h


def _path_scan(xg4, h04, wc4, b4, p4_blk):
    n4 = h04.shape[0]
    grid = n4 // p4_blk
    return pl.pallas_call(
        _path_scan_body,
        grid=(grid,),
        in_specs=[
            pl.BlockSpec((L, p4_blk, G3), lambda i: (0, i, 0)),
            pl.BlockSpec((p4_blk, G3), lambda i: (i, 0)),
            pl.BlockSpec((PK * (D + H), GW), lambda i: (0, 0)),
            pl.BlockSpec((1, GW), lambda i: (0, 0)),
        ],
        out_specs=pl.BlockSpec((L, p4_blk, G3), lambda i: (0, i, 0)),
        out_shape=jax.ShapeDtypeStruct((L, n4, G3), jnp.float32),
    )(xg4, h04, wc4, b4)


def _link_update_body(agg_ref, h_ref, wx_ref, wh_ref, b_ref, out_ref,
                      out16_ref):
    x = agg_ref[0] + agg_ref[1]
    h = h_ref[...]
    g = (jnp.dot(x, wx_ref[...], preferred_element_type=jnp.float32)
         + jnp.dot(h, wh_ref[...], preferred_element_type=jnp.float32)
         + b_ref[...])
    new_h = _gru_gates4(g, h)
    out_ref[...] = new_h
    out16_ref[...] = new_h.astype(jnp.bfloat16)


def _link_update(agg2, link_state4, wx4, wh4, b4):
    n4 = link_state4.shape[0]
    return pl.pallas_call(
        _link_update_body,
        out_shape=(jax.ShapeDtypeStruct((n4, G3), jnp.float32),
                   jax.ShapeDtypeStruct((n4, G3), jnp.bfloat16)),
    )(agg2, link_state4, wx4, wh4, b4)


def _block_diag4(W):
    a, b = W.shape
    out = jnp.einsum('rc,jk->jrkc', W, jnp.eye(PK, dtype=W.dtype))
    return out.reshape(PK * a, PK * b)


def _readout_body(ps_ref, w1_ref, b1_ref, w2_ref, b2_ref, w3_ref, b3_ref,
                  y_ref):
    x1 = jax.nn.relu(jnp.dot(ps_ref[...], w1_ref[...],
                             preferred_element_type=jnp.float32) + b1_ref[...])
    x2 = jax.nn.relu(jnp.dot(x1, w2_ref[...],
                             preferred_element_type=jnp.float32) + b2_ref[...])
    y_ref[...] = jnp.dot(x2, w3_ref[...],
                         preferred_element_type=jnp.float32) + b3_ref[...]


def _readout(ps4, W1, b1, W2, b2, W3, b3):
    n4 = ps4.shape[0]
    R = W1.shape[0]
    y4 = pl.pallas_call(
        _readout_body,
        out_shape=jax.ShapeDtypeStruct((n4, PK), jnp.float32),
    )(ps4, _block_diag4(W1.T), jnp.tile(b1, PK).reshape(1, PK * R),
      _block_diag4(W2.T), jnp.tile(b2, PK).reshape(1, PK * R),
      _block_diag4(W3.T), jnp.tile(b3, PK).reshape(1, PK))
    return y4.reshape(n4 * PK, 1)


def kernel(links, paths, sequences, link_capacity, bandwith, n_links, n_paths,
           Wih_p, Whh_p, bih_p, bhh_p, Wih_l, Whh_l, bih_l, bhh_l,
           W1, b1, W2, b2, W3, b3):
    n_links_s = link_capacity.shape[0]
    n_paths_s = bandwith.shape[0]
    E = links.shape[0]

    link_state = jnp.concatenate(
        [link_capacity[:, None],
         jnp.zeros((n_links_s, D - 1), jnp.float32)], axis=1)
    path_state4 = jnp.concatenate(
        [bandwith[:, None],
         jnp.zeros((n_paths_s, D - 1), jnp.float32)],
        axis=1).reshape(n_paths_s // PK, G3)

    wx_p, wh_p, b_p = _pack_gru_weights(Wih_p, Whh_p, bih_p, bhh_p)
    wc_p = jnp.concatenate([wx_p, wh_p], axis=0)
    wx_l, wh_l, b_l = _pack_gru_weights(Wih_l, Whh_l, bih_l, bhh_l)

    p4_blk = 1000

    links_tm = links.reshape(n_paths_s, L).T.reshape(E // GSZ, GSZ)
    zeros_nl = jnp.zeros((n_links_s, D), jnp.float32)

    link_state16 = link_state.astype(jnp.bfloat16)
    for _ in range(T):
        xg = _sc_gather(link_state16, links_tm, E)
        xg4 = xg.reshape(L, n_paths_s // PK, G3)
        hs4 = _path_scan(xg4, path_state4, wc_p, b_p, p4_blk)
        path_state4 = hs4[L - 1]
        msgs = hs4.reshape(E, H)
        agg2 = _sc_scatter_add(msgs, links_tm, zeros_nl, n_links_s)
        link_state4, link_state4_16 = _link_update(
            agg2.reshape(NC, n_links_s // PK, G3),
            link_state.reshape(n_links_s // PK, G3), wx_l, wh_l, b_l)
        link_state = link_state4.reshape(n_links_s, H)
        link_state16 = link_state4_16.reshape(n_links_s, H)

    return _readout(path_state4, W1, b1, W2, b2, W3, b3)


# speedup vs baseline: 1.7010x; 1.7010x over previous
import functools

import jax
import jax.numpy as jnp
from jax import lax
from jax.experimental import pallas as pl
from jax.experimental.pallas import tpu as pltpu
from jax.experimental.pallas import tpu_sc as plsc

D = 32
H = 32
L = 16
T = 8
G3 = 128
PK = 4
GW = G3 * PK

NC = 2
NS = 16
NW = NC * NS
GSZ = 125
CHUNK_G = 8
CHUNK = GSZ * CHUNK_G

_MESH = plsc.VectorSubcoreMesh(core_axis_name="c", subcore_axis_name="s")


def _sc_gather(table, idx_groups, E):
    n_groups = idx_groups.shape[0]
    gpw = n_groups // NW
    chunks = gpw // CHUNK_G

    @functools.partial(
        pl.kernel,
        out_type=jax.ShapeDtypeStruct((E, D), jnp.float32),
        mesh=_MESH,
        scratch_types=[
            pltpu.VMEM((gpw, GSZ), jnp.int32),
            pltpu.VMEM((CHUNK, D), jnp.float32),
            pltpu.VMEM((CHUNK, D), jnp.float32),
            pltpu.SemaphoreType.DMA,
            pltpu.SemaphoreType.DMA,
            pltpu.SemaphoreType.DMA,
            pltpu.SemaphoreType.DMA,
        ],
        compiler_params=pltpu.CompilerParams(use_tc_tiling_on_sc=False),
    )
    def k(table_hbm, idx_hbm, out_hbm, idx_v, rows0, rows1,
          gsem0, gsem1, wsem0, wsem1):
        cid = lax.axis_index("c")
        sid = lax.axis_index("s")
        wid = sid * NC + cid
        obase = wid * gpw * GSZ
        pltpu.sync_copy(idx_hbm.at[pl.ds(wid * gpw, gpw)], idx_v)
        bufs = (rows0, rows1)
        gsems = (gsem0, gsem1)
        wsems = (wsem0, wsem1)

        def fire(c, buf, sem):
            for g in range(CHUNK_G):
                pltpu.async_copy(table_hbm.at[idx_v.at[c * CHUNK_G + g]],
                                 buf.at[pl.ds(g * GSZ, GSZ)], sem)

        def drain(buf, sem):
            pltpu.make_async_copy(out_hbm.at[pl.ds(0, CHUNK)], buf,
                                  sem).wait()

        def drain_write(buf, sem):
            pltpu.make_async_copy(buf, out_hbm.at[pl.ds(0, CHUNK)],
                                  sem).wait()

        fire(0, bufs[0], gsems[0])

        def body(c2, _):
            for b in range(2):
                c = 2 * c2 + b
                o = 1 - b
                nxt = c + 1

                @pl.when(nxt < chunks)
                def _():
                    @pl.when(nxt >= 2)
                    def _():
                        drain_write(bufs[o], wsems[o])
                    fire(nxt, bufs[o], gsems[o])

                drain(bufs[b], gsems[b])
                pltpu.async_copy(bufs[b], out_hbm.at[pl.ds(obase + c * CHUNK,
                                                           CHUNK)], wsems[b])
            return 0

        lax.fori_loop(0, chunks // 2, body, 0)
        drain_write(bufs[0], wsems[0])
        drain_write(bufs[1], wsems[1])

    return k(table, idx_groups)


def _sc_scatter_add(msgs, idx_groups, zeros, n_links):
    n_groups = idx_groups.shape[0]
    gpw = n_groups // NW
    chunks = gpw // CHUNK_G
    rows_per_sub = n_links // NS

    @functools.partial(
        pl.kernel,
        out_type=jax.ShapeDtypeStruct((NC, n_links, D), jnp.float32),
        mesh=_MESH,
        scratch_types=[
            pltpu.VMEM((gpw, GSZ), jnp.int32),
            pltpu.VMEM((CHUNK, D), jnp.float32),
            pltpu.VMEM((CHUNK, D), jnp.float32),
            pltpu.VMEM_SHARED((n_links, D), jnp.float32),
            pltpu.SemaphoreType.DMA,
            pltpu.SemaphoreType.DMA,
            pltpu.SemaphoreType.DMA,
            pltpu.SemaphoreType.DMA,
        ],
        compiler_params=pltpu.CompilerParams(use_tc_tiling_on_sc=False),
    )
    def k(msgs_hbm, idx_hbm, zeros_hbm, out_hbm, idx_v, rows0, rows1, acc_sh,
          rsem0, rsem1, ssem0, ssem1):
        cid = lax.axis_index("c")
        sid = lax.axis_index("s")
        wid = sid * NC + cid
        pltpu.sync_copy(zeros_hbm.at[pl.ds(sid * rows_per_sub, rows_per_sub)],
                        acc_sh.at[pl.ds(sid * rows_per_sub, rows_per_sub)])
        pltpu.sync_copy(idx_hbm.at[pl.ds(wid * gpw, gpw)], idx_v)
        plsc.subcore_barrier()
        bufs = (rows0, rows1)
        rsems = (rsem0, rsem1)
        ssems = (ssem0, ssem1)
        mbase = wid * gpw * GSZ

        def fire_sadds(c, buf, sem):
            for g in range(CHUNK_G):
                pltpu.async_copy(buf.at[pl.ds(g * GSZ, GSZ)],
                                 acc_sh.at[idx_v.at[c * CHUNK_G + g]],
                                 sem, add=True)

        def drain_chunk(buf, sem):
            pltpu.make_async_copy(msgs_hbm.at[pl.ds(0, CHUNK)], buf,
                                  sem).wait()

        pltpu.async_copy(msgs_hbm.at[pl.ds(mbase, CHUNK)], bufs[0], rsems[0])

        def body(c2, _):
            for b in range(2):
                c = 2 * c2 + b
                o = 1 - b
                nxt = c + 1

                @pl.when(nxt < chunks)
                def _():
                    @pl.when(nxt >= 2)
                    def _():
                        drain_chunk(bufs[o], ssems[o])
                    pltpu.async_copy(
                        msgs_hbm.at[pl.ds(mbase + nxt * CHUNK, CHUNK)],
                        bufs[o], rsems[o])

                drain_chunk(bufs[b], rsems[b])
                fire_sadds(c, bufs[b], ssems[b])
            return 0

        lax.fori_loop(0, chunks // 2, body, 0)
        drain_chunk(bufs[0], ssems[0])
        drain_chunk(bufs[1], ssems[1])
        plsc.subcore_barrier()
        pltpu.sync_copy(acc_sh.at[pl.ds(sid * rows_per_sub, rows_per_sub)],
                        out_hbm.at[cid, pl.ds(sid * rows_per_sub,
                                              rows_per_sub)])

    return k(msgs, idx_groups, zeros)


def _pack_gru_weights(Wih, Whh, bih, bhh):
    Wx = jnp.concatenate(
        [Wih[0:H].T, Wih[H:2 * H].T, Wih[2 * H:3 * H].T,
         jnp.zeros((D, H), Wih.dtype)], axis=1)
    Wh = jnp.concatenate(
        [Whh[0:H].T, Whh[H:2 * H].T, jnp.zeros((H, H), Whh.dtype),
         Whh[2 * H:3 * H].T], axis=1)
    b = jnp.concatenate(
        [bih[0:H] + bhh[0:H], bih[H:2 * H] + bhh[H:2 * H],
         bih[2 * H:3 * H], bhh[2 * H:3 * H]])
    scale = jnp.concatenate([jnp.full((2 * H,), 0.5, Wx.dtype),
                             jnp.ones((2 * H,), Wx.dtype)])
    Wx = Wx * scale
    Wh = Wh * scale
    b = b * scale
    eye = jnp.eye(PK, dtype=Wx.dtype)
    Wx4 = jnp.einsum('rgc,jk->jrgkc', Wx.reshape(D, 4, H), eye)
    Wx4 = Wx4.reshape(PK * D, GW)
    Wh4 = jnp.einsum('rgc,jk->jrgkc', Wh.reshape(H, 4, H), eye)
    Wh4 = Wh4.reshape(PK * H, GW)
    b4 = jnp.tile(b.reshape(4, 1, H), (1, PK, 1)).reshape(1, GW)
    return Wx4, Wh4, b4


def _gru_gates4(g4, h4):
    rz = jnp.tanh(g4[:, 0:2 * G3]) * 0.5 + 0.5
    r = rz[:, 0:G3]
    z = rz[:, G3:2 * G3]
    n = jnp.tanh(g4[:, 2 * G3:3 * G3] + r * g4[:, 3 * G3:4 * G3])
    return n + z * (h4 - n)


def _path_scan_body(xg_ref, h0_ref, wc_ref, b_ref, hs_ref):
    h = h0_ref[...]
    wc = wc_ref[...]
    b = b_ref[...]
    for s in range(L):
        xh = jnp.concatenate([xg_ref[s], h], axis=1)
        g = jnp.dot(xh, wc, preferred_element_type=jnp.float32) + b
        h = _gru_gates4(g, h)
        hs_ref[s] = h


def _path_scan(xg4, h04, wc4, b4, p4_blk):
    n4 = h04.shape[0]
    grid = n4 // p4_blk
    return pl.pallas_call(
        _path_scan_body,
        grid=(grid,),
        in_specs=[
            pl.BlockSpec((L, p4_blk, G3), lambda i: (0, i, 0)),
            pl.BlockSpec((p4_blk, G3), lambda i: (i, 0)),
            pl.BlockSpec((PK * (D + H), GW), lambda i: (0, 0)),
            pl.BlockSpec((1, GW), lambda i: (0, 0)),
        ],
        out_specs=pl.BlockSpec((L, p4_blk, G3), lambda i: (0, i, 0)),
        out_shape=jax.ShapeDtypeStruct((L, n4, G3), jnp.float32),
    )(xg4, h04, wc4, b4)


def _link_update_body(agg_ref, h_ref, wx_ref, wh_ref, b_ref, out_ref):
    x = agg_ref[0] + agg_ref[1]
    h = h_ref[...]
    g = (jnp.dot(x, wx_ref[...], preferred_element_type=jnp.float32)
         + jnp.dot(h, wh_ref[...], preferred_element_type=jnp.float32)
         + b_ref[...])
    out_ref[...] = _gru_gates4(g, h)


def _link_update(agg2, link_state4, wx4, wh4, b4):
    n4 = link_state4.shape[0]
    return pl.pallas_call(
        _link_update_body,
        out_shape=jax.ShapeDtypeStruct((n4, G3), jnp.float32),
    )(agg2, link_state4, wx4, wh4, b4)


def _block_diag4(W):
    a, b = W.shape
    out = jnp.einsum('rc,jk->jrkc', W, jnp.eye(PK, dtype=W.dtype))
    return out.reshape(PK * a, PK * b)


def _readout_body(ps_ref, w1_ref, b1_ref, w2_ref, b2_ref, w3_ref, b3_ref,
                  y_ref):
    x1 = jax.nn.relu(jnp.dot(ps_ref[...], w1_ref[...],
                             preferred_element_type=jnp.float32) + b1_ref[...])
    x2 = jax.nn.relu(jnp.dot(x1, w2_ref[...],
                             preferred_element_type=jnp.float32) + b2_ref[...])
    y_ref[...] = jnp.dot(x2, w3_ref[...],
                         preferred_element_type=jnp.float32) + b3_ref[...]


def _readout(ps4, W1, b1, W2, b2, W3, b3):
    n4 = ps4.shape[0]
    R = W1.shape[0]
    y4 = pl.pallas_call(
        _readout_body,
        out_shape=jax.ShapeDtypeStruct((n4, PK), jnp.float32),
    )(ps4, _block_diag4(W1.T), jnp.tile(b1, PK).reshape(1, PK * R),
      _block_diag4(W2.T), jnp.tile(b2, PK).reshape(1, PK * R),
      _block_diag4(W3.T), jnp.tile(b3, PK).reshape(1, PK))
    return y4.reshape(n4 * PK, 1)


def kernel(links, paths, sequences, link_capacity, bandwith, n_links, n_paths,
           Wih_p, Whh_p, bih_p, bhh_p, Wih_l, Whh_l, bih_l, bhh_l,
           W1, b1, W2, b2, W3, b3):
    n_links_s = link_capacity.shape[0]
    n_paths_s = bandwith.shape[0]
    E = links.shape[0]

    link_state = jnp.concatenate(
        [link_capacity[:, None],
         jnp.zeros((n_links_s, D - 1), jnp.float32)], axis=1)
    path_state4 = jnp.concatenate(
        [bandwith[:, None],
         jnp.zeros((n_paths_s, D - 1), jnp.float32)],
        axis=1).reshape(n_paths_s // PK, G3)

    wx_p, wh_p, b_p = _pack_gru_weights(Wih_p, Whh_p, bih_p, bhh_p)
    wc_p = jnp.concatenate([wx_p, wh_p], axis=0)
    wx_l, wh_l, b_l = _pack_gru_weights(Wih_l, Whh_l, bih_l, bhh_l)

    p4_blk = 1000

    links_tm = links.reshape(n_paths_s, L).T.reshape(E // GSZ, GSZ)
    zeros_nl = jnp.zeros((n_links_s, D), jnp.float32)

    for _ in range(T):
        xg = _sc_gather(link_state, links_tm, E)
        xg4 = xg.reshape(L, n_paths_s // PK, G3)
        hs4 = _path_scan(xg4, path_state4, wc_p, b_p, p4_blk)
        path_state4 = hs4[L - 1]
        msgs = hs4.reshape(E, H)
        agg2 = _sc_scatter_add(msgs, links_tm, zeros_nl, n_links_s)
        link_state4 = _link_update(
            agg2.reshape(NC, n_links_s // PK, G3),
            link_state.reshape(n_links_s // PK, G3), wx_l, wh_l, b_l)
        link_state = link_state4.reshape(n_links_s, H)

    return _readout(path_state4, W1, b1, W2, b2, W3, b3)
